# Initial kernel scaffold; baseline (speedup 1.0000x reference)
import jax
import jax.numpy as jnp
from jax.experimental import pallas as pl


def kernel(x, emb, W1, b1, W2, b2, W3, b3):
    raise NotImplementedError("write your pallas kernel here")


# speedup vs baseline: 3.6173x; 3.6173x over previous
import functools

import jax
import jax.numpy as jnp
from jax import lax
from jax.experimental import pallas as pl
from jax.experimental.pallas import tpu as pltpu
from jax.experimental.pallas import tpu_sc as plsc

_NC = 2
_NS = 16
_NW = _NC * _NS

_GATHER_CHUNK = 128


def _make_gather(n_rows, d):
    rows_per_w = n_rows // _NW
    n_chunks = rows_per_w // _GATHER_CHUNK
    mesh = plsc.VectorSubcoreMesh(core_axis_name="c", subcore_axis_name="s")

    @functools.partial(
        pl.kernel,
        mesh=mesh,
        out_type=jax.ShapeDtypeStruct((n_rows, d), jnp.float32),
        scratch_types=[
            pltpu.VMEM((_GATHER_CHUNK,), jnp.int32),
            pltpu.VMEM((_GATHER_CHUNK, d), jnp.float32),
            pltpu.SemaphoreType.DMA,
        ],
    )
    def gather_k(idx_hbm, table_hbm, out_hbm, idx_v, rows_v, sem):
        wid = lax.axis_index("s") * _NC + lax.axis_index("c")
        base = wid * rows_per_w
        for c in range(n_chunks):
            off = base + c * _GATHER_CHUNK
            pltpu.sync_copy(idx_hbm.at[pl.ds(off, _GATHER_CHUNK)], idx_v)
            pltpu.async_copy(table_hbm.at[idx_v], rows_v, sem).wait()
            pltpu.sync_copy(rows_v, out_hbm.at[pl.ds(off, _GATHER_CHUNK)])

    return gather_k


def _mlp_body(g_ref, w1_ref, b1_ref, w2_ref, b2_ref, w3_ref, b3_ref, out_ref):
    h = jnp.dot(g_ref[:], w1_ref[:], preferred_element_type=jnp.float32)
    h = jnp.maximum(h + b1_ref[:], 0.0)
    h = jnp.dot(h, w2_ref[:], preferred_element_type=jnp.float32)
    h = jnp.maximum(h + b2_ref[:], 0.0)
    out_ref[:] = jnp.sum(h * w3_ref[:], axis=1, keepdims=True) + b3_ref[:]


def _mlp(g, w1, b1, w2, b2, w3t, b3, block_m=256):
    batch, k1 = g.shape
    hidden = w1.shape[1]
    return pl.pallas_call(
        _mlp_body,
        grid=(batch // block_m,),
        in_specs=[
            pl.BlockSpec((block_m, k1), lambda i: (i, 0)),
            pl.BlockSpec((k1, hidden), lambda i: (0, 0)),
            pl.BlockSpec((1, hidden), lambda i: (0, 0)),
            pl.BlockSpec((hidden, hidden), lambda i: (0, 0)),
            pl.BlockSpec((1, hidden), lambda i: (0, 0)),
            pl.BlockSpec((1, hidden), lambda i: (0, 0)),
            pl.BlockSpec((1, 1), lambda i: (0, 0)),
        ],
        out_specs=pl.BlockSpec((block_m, 1), lambda i: (i, 0)),
        out_shape=jax.ShapeDtypeStruct((batch, 1), jnp.float32),
    )(g, w1, b1, w2, b2, w3t, b3)


def kernel(x, emb, W1, b1, W2, b2, W3, b3):
    batch = x.shape[0]
    d = emb.shape[1]
    idx_flat = x.astype(jnp.int32).reshape(-1)
    gathered = _make_gather(idx_flat.shape[0], d)(idx_flat, emb)
    g = gathered.reshape(batch, 2 * d)
    hidden = W1.shape[1]
    return _mlp(
        g,
        W1,
        b1.reshape(1, hidden),
        W2,
        b2.reshape(1, hidden),
        W3.reshape(1, hidden),
        b3.reshape(1, 1),
    )


# speedup vs baseline: 3.6214x; 1.0011x over previous
import functools

import jax
import jax.numpy as jnp
from jax import lax
from jax.experimental import pallas as pl
from jax.experimental.pallas import tpu as pltpu
from jax.experimental.pallas import tpu_sc as plsc

_NC = 2
_NS = 16
_NW = _NC * _NS

_GATHER_CHUNK = 128


def _make_gather(n_rows, d):
    rows_per_w = n_rows // _NW
    n_chunks = rows_per_w // _GATHER_CHUNK
    mesh = plsc.VectorSubcoreMesh(core_axis_name="c", subcore_axis_name="s")

    @functools.partial(
        pl.kernel,
        mesh=mesh,
        out_type=jax.ShapeDtypeStruct((n_rows, d), jnp.float32),
        scratch_types=[
            pltpu.VMEM((_GATHER_CHUNK,), jnp.int32),
            pltpu.VMEM((_GATHER_CHUNK, d), jnp.float32),
            pltpu.SemaphoreType.DMA,
        ],
    )
    def gather_k(idx_hbm, table_hbm, out_hbm, idx_v, rows_v, sem):
        wid = lax.axis_index("s") * _NC + lax.axis_index("c")
        base = wid * rows_per_w
        for c in range(n_chunks):
            off = base + c * _GATHER_CHUNK
            pltpu.sync_copy(idx_hbm.at[pl.ds(off, _GATHER_CHUNK)], idx_v)
            pltpu.async_copy(table_hbm.at[idx_v], rows_v, sem).wait()
            pltpu.sync_copy(rows_v, out_hbm.at[pl.ds(off, _GATHER_CHUNK)])

    return gather_k


def _mlp_body(g_ref, w1_ref, b1_ref, w2_ref, b2_ref, w3_ref, b3_ref, out_ref):
    g = g_ref[:].astype(jnp.bfloat16)
    h = jnp.dot(g, w1_ref[:].astype(jnp.bfloat16),
                preferred_element_type=jnp.float32)
    h = jnp.maximum(h + b1_ref[:], 0.0).astype(jnp.bfloat16)
    h = jnp.dot(h, w2_ref[:].astype(jnp.bfloat16),
                preferred_element_type=jnp.float32)
    h = jnp.maximum(h + b2_ref[:], 0.0)
    out_ref[:] = jnp.sum(h * w3_ref[:], axis=1, keepdims=True) + b3_ref[:]


def _mlp(g, w1, b1, w2, b2, w3t, b3, block_m=256):
    batch, k1 = g.shape
    hidden = w1.shape[1]
    return pl.pallas_call(
        _mlp_body,
        grid=(batch // block_m,),
        in_specs=[
            pl.BlockSpec((block_m, k1), lambda i: (i, 0)),
            pl.BlockSpec((k1, hidden), lambda i: (0, 0)),
            pl.BlockSpec((1, hidden), lambda i: (0, 0)),
            pl.BlockSpec((hidden, hidden), lambda i: (0, 0)),
            pl.BlockSpec((1, hidden), lambda i: (0, 0)),
            pl.BlockSpec((1, hidden), lambda i: (0, 0)),
            pl.BlockSpec((1, 1), lambda i: (0, 0)),
        ],
        out_specs=pl.BlockSpec((block_m, 1), lambda i: (i, 0)),
        out_shape=jax.ShapeDtypeStruct((batch, 1), jnp.float32),
    )(g, w1, b1, w2, b2, w3t, b3)


def kernel(x, emb, W1, b1, W2, b2, W3, b3):
    batch = x.shape[0]
    d = emb.shape[1]
    idx_flat = x.astype(jnp.int32).reshape(-1)
    gathered = _make_gather(idx_flat.shape[0], d)(idx_flat, emb)
    g = gathered.reshape(batch, 2 * d)
    hidden = W1.shape[1]
    return _mlp(
        g,
        W1,
        b1.reshape(1, hidden),
        W2,
        b2.reshape(1, hidden),
        W3.reshape(1, hidden),
        b3.reshape(1, 1),
    )


# speedup vs baseline: 3.9565x; 1.0925x over previous
import functools

import jax
import jax.numpy as jnp
from jax import lax
from jax.experimental import pallas as pl
from jax.experimental.pallas import tpu as pltpu
from jax.experimental.pallas import tpu_sc as plsc

_NC = 2
_NS = 16
_NW = _NC * _NS

_GATHER_CHUNK = 128


def _make_gather(n_rows, d):
    rows_per_w = n_rows // _NW
    n_chunks = rows_per_w // _GATHER_CHUNK
    mesh = plsc.VectorSubcoreMesh(core_axis_name="c", subcore_axis_name="s")

    @functools.partial(
        pl.kernel,
        mesh=mesh,
        out_type=jax.ShapeDtypeStruct((n_rows, d), jnp.float32),
        scratch_types=[
            pltpu.VMEM((_GATHER_CHUNK,), jnp.int32),
            pltpu.VMEM((_GATHER_CHUNK, d), jnp.float32),
            pltpu.SemaphoreType.DMA,
        ],
    )
    def gather_k(idx_hbm, table_hbm, out_hbm, idx_v, rows_v, sem):
        wid = lax.axis_index("s") * _NC + lax.axis_index("c")
        base = wid * rows_per_w
        for c in range(n_chunks):
            off = base + c * _GATHER_CHUNK
            pltpu.sync_copy(idx_hbm.at[pl.ds(off, _GATHER_CHUNK)], idx_v)
            pltpu.async_copy(table_hbm.at[idx_v], rows_v, sem).wait()
            pltpu.sync_copy(rows_v, out_hbm.at[pl.ds(off, _GATHER_CHUNK)])

    return gather_k


def _mlp_body(g_ref, w1_ref, b1_ref, w2_ref, b2_ref, w3_ref, b3_ref, out_ref):
    g = g_ref[:].astype(jnp.bfloat16)
    h = jnp.dot(g, w1_ref[:], preferred_element_type=jnp.float32)
    h = jnp.maximum(h + b1_ref[:], 0.0).astype(jnp.bfloat16)
    h = jnp.dot(h, w2_ref[:], preferred_element_type=jnp.float32)
    h = jnp.maximum(h + b2_ref[:], 0.0)
    out_ref[:] = jnp.sum(h * w3_ref[:], axis=1, keepdims=True) + b3_ref[:]


def _mlp(g, w1, b1, w2, b2, w3t, b3, block_m=512):
    batch, k1 = g.shape
    hidden = w1.shape[1]
    return pl.pallas_call(
        _mlp_body,
        grid=(batch // block_m,),
        in_specs=[
            pl.BlockSpec((block_m, k1), lambda i: (i, 0)),
            pl.BlockSpec((k1, hidden), lambda i: (0, 0)),
            pl.BlockSpec((1, hidden), lambda i: (0, 0)),
            pl.BlockSpec((hidden, hidden), lambda i: (0, 0)),
            pl.BlockSpec((1, hidden), lambda i: (0, 0)),
            pl.BlockSpec((1, hidden), lambda i: (0, 0)),
            pl.BlockSpec((1, 1), lambda i: (0, 0)),
        ],
        out_specs=pl.BlockSpec((block_m, 1), lambda i: (i, 0)),
        out_shape=jax.ShapeDtypeStruct((batch, 1), jnp.float32),
    )(g, w1, b1, w2, b2, w3t, b3)


def kernel(x, emb, W1, b1, W2, b2, W3, b3):
    batch = x.shape[0]
    d = emb.shape[1]
    idx_flat = x.astype(jnp.int32).reshape(-1)
    gathered = _make_gather(idx_flat.shape[0], d)(idx_flat, emb)
    g = gathered.reshape(batch, 2 * d)
    hidden = W1.shape[1]
    return _mlp(
        g,
        W1.astype(jnp.bfloat16),
        b1.reshape(1, hidden),
        W2.astype(jnp.bfloat16),
        b2.reshape(1, hidden),
        W3.reshape(1, hidden),
        b3.reshape(1, 1),
    )


# speedup vs baseline: 4.0605x; 1.0263x over previous
import functools

import jax
import jax.numpy as jnp
from jax import lax
from jax.experimental import pallas as pl
from jax.experimental.pallas import tpu as pltpu
from jax.experimental.pallas import tpu_sc as plsc

_NC = 2
_NS = 16
_NW = _NC * _NS

_GATHER_CHUNK = 128


def _make_gather(n_rows, d):
    rows_per_w = n_rows // _NW
    n_chunks = rows_per_w // _GATHER_CHUNK
    mesh = plsc.VectorSubcoreMesh(core_axis_name="c", subcore_axis_name="s")

    @functools.partial(
        pl.kernel,
        mesh=mesh,
        out_type=jax.ShapeDtypeStruct((n_rows, d), jnp.float32),
        scratch_types=[
            pltpu.VMEM((_GATHER_CHUNK,), jnp.int32),
            pltpu.VMEM((_GATHER_CHUNK, d), jnp.float32),
            pltpu.SemaphoreType.DMA,
        ],
    )
    def gather_k(idx_hbm, table_hbm, out_hbm, idx_v, rows_v, sem):
        wid = lax.axis_index("s") * _NC + lax.axis_index("c")
        base = wid * rows_per_w
        for c in range(n_chunks):
            off = base + c * _GATHER_CHUNK
            pltpu.sync_copy(idx_hbm.at[pl.ds(off, _GATHER_CHUNK)], idx_v)
            pltpu.async_copy(table_hbm.at[idx_v], rows_v, sem).wait()
            pltpu.sync_copy(rows_v, out_hbm.at[pl.ds(off, _GATHER_CHUNK)])

    return gather_k


def _mlp_body(g_ref, w1_ref, b1_ref, w2_ref, b2_ref, w3_ref, b3_ref, out_ref):
    g = g_ref[:].astype(jnp.bfloat16)
    h = jnp.dot(g, w1_ref[:], preferred_element_type=jnp.float32)
    h = jnp.maximum(h + b1_ref[:], 0.0).astype(jnp.bfloat16)
    h = jnp.dot(h, w2_ref[:], preferred_element_type=jnp.float32)
    h = jnp.maximum(h + b2_ref[:], 0.0)
    out_ref[:] = jnp.sum(h * w3_ref[:], axis=1, keepdims=True) + b3_ref[:]


def _mlp(g, w1, b1, w2, b2, w3t, b3, block_m=512):
    batch, k1 = g.shape
    hidden = w1.shape[1]
    return pl.pallas_call(
        _mlp_body,
        grid=(batch // block_m,),
        in_specs=[
            pl.BlockSpec((block_m, k1), lambda i: (i, 0)),
            pl.BlockSpec((k1, hidden), lambda i: (0, 0)),
            pl.BlockSpec((1, hidden), lambda i: (0, 0)),
            pl.BlockSpec((hidden, hidden), lambda i: (0, 0)),
            pl.BlockSpec((1, hidden), lambda i: (0, 0)),
            pl.BlockSpec((1, hidden), lambda i: (0, 0)),
            pl.BlockSpec((1, 1), lambda i: (0, 0)),
        ],
        out_specs=pl.BlockSpec((block_m, 1), lambda i: (i, 0)),
        out_shape=jax.ShapeDtypeStruct((batch, 1), jnp.float32),
    )(g, w1, b1, w2, b2, w3t, b3)


_N_CHUNKS = 2


def kernel(x, emb, W1, b1, W2, b2, W3, b3):
    batch = x.shape[0]
    d = emb.shape[1]
    hidden = W1.shape[1]
    idx_flat = x.astype(jnp.int32).reshape(-1)
    n_idx = idx_flat.shape[0]
    chunk_idx = n_idx // _N_CHUNKS
    gather_fn = _make_gather(chunk_idx, d)
    w1 = W1.astype(jnp.bfloat16)
    w2 = W2.astype(jnp.bfloat16)
    b1r = b1.reshape(1, hidden)
    b2r = b2.reshape(1, hidden)
    w3t = W3.reshape(1, hidden)
    b3r = b3.reshape(1, 1)
    gs = [
        gather_fn(lax.dynamic_slice_in_dim(idx_flat, c * chunk_idx, chunk_idx), emb)
        for c in range(_N_CHUNKS)
    ]
    outs = [
        _mlp(g.reshape(chunk_idx // 2, 2 * d), w1, b1r, w2, b2r, w3t, b3r)
        for g in gs
    ]
    return jnp.concatenate(outs, axis=0)


# speedup vs baseline: 4.1726x; 1.0276x over previous
import functools

import jax
import jax.numpy as jnp
from jax import lax
from jax.experimental import pallas as pl
from jax.experimental.pallas import tpu as pltpu
from jax.experimental.pallas import tpu_sc as plsc

_NC = 2
_NS = 16
_NW = _NC * _NS

_GATHER_CHUNK = 128


def _make_gather(n_rows, d):
    rows_per_w = n_rows // _NW
    n_chunks = rows_per_w // _GATHER_CHUNK
    mesh = plsc.VectorSubcoreMesh(core_axis_name="c", subcore_axis_name="s")

    @functools.partial(
        pl.kernel,
        mesh=mesh,
        out_type=jax.ShapeDtypeStruct((n_rows, d), jnp.float32),
        scratch_types=[
            pltpu.VMEM((_GATHER_CHUNK,), jnp.int32),
            pltpu.VMEM((_GATHER_CHUNK, d), jnp.float32),
            pltpu.SemaphoreType.DMA,
        ],
    )
    def gather_k(idx_hbm, table_hbm, out_hbm, idx_v, rows_v, sem):
        wid = lax.axis_index("s") * _NC + lax.axis_index("c")
        base = wid * rows_per_w
        for c in range(n_chunks):
            off = base + c * _GATHER_CHUNK
            pltpu.sync_copy(idx_hbm.at[pl.ds(off, _GATHER_CHUNK)], idx_v)
            pltpu.async_copy(table_hbm.at[idx_v], rows_v, sem).wait()
            pltpu.sync_copy(rows_v, out_hbm.at[pl.ds(off, _GATHER_CHUNK)])

    return gather_k


def _mlp_body(g_ref, w1_ref, b1_ref, w2_ref, b2_ref, w3_ref, b3_ref, out_ref):
    g = g_ref[:].astype(jnp.bfloat16)
    h = jnp.dot(g, w1_ref[:], preferred_element_type=jnp.float32)
    h = jnp.maximum(h + b1_ref[:], 0.0).astype(jnp.bfloat16)
    h = jnp.dot(h, w2_ref[:], preferred_element_type=jnp.float32)
    h = jnp.maximum(h + b2_ref[:], 0.0)
    out_ref[:] = jnp.sum(h * w3_ref[:], axis=1, keepdims=True) + b3_ref[:]


def _mlp(g, w1, b1, w2, b2, w3t, b3, block_m=1024):
    batch, k1 = g.shape
    hidden = w1.shape[1]
    return pl.pallas_call(
        _mlp_body,
        grid=(batch // block_m,),
        in_specs=[
            pl.BlockSpec((block_m, k1), lambda i: (i, 0)),
            pl.BlockSpec((k1, hidden), lambda i: (0, 0)),
            pl.BlockSpec((1, hidden), lambda i: (0, 0)),
            pl.BlockSpec((hidden, hidden), lambda i: (0, 0)),
            pl.BlockSpec((1, hidden), lambda i: (0, 0)),
            pl.BlockSpec((1, hidden), lambda i: (0, 0)),
            pl.BlockSpec((1, 1), lambda i: (0, 0)),
        ],
        out_specs=pl.BlockSpec((block_m, 1), lambda i: (i, 0)),
        out_shape=jax.ShapeDtypeStruct((batch, 1), jnp.float32),
    )(g, w1, b1, w2, b2, w3t, b3)


_N_CHUNKS = 2


def kernel(x, emb, W1, b1, W2, b2, W3, b3):
    batch = x.shape[0]
    d = emb.shape[1]
    hidden = W1.shape[1]
    idx_flat = x.astype(jnp.int32).reshape(-1)
    n_idx = idx_flat.shape[0]
    chunk_idx = n_idx // _N_CHUNKS
    gather_fn = _make_gather(chunk_idx, d)
    w1 = W1.astype(jnp.bfloat16)
    w2 = W2.astype(jnp.bfloat16)
    b1r = b1.reshape(1, hidden)
    b2r = b2.reshape(1, hidden)
    w3t = W3.reshape(1, hidden)
    b3r = b3.reshape(1, 1)
    gs = [
        gather_fn(lax.dynamic_slice_in_dim(idx_flat, c * chunk_idx, chunk_idx), emb)
        for c in range(_N_CHUNKS)
    ]
    outs = [
        _mlp(g.reshape(chunk_idx // 2, 2 * d), w1, b1r, w2, b2r, w3t, b3r)
        for g in gs
    ]
    return jnp.concatenate(outs, axis=0)
